# Initial kernel scaffold; baseline (speedup 1.0000x reference)
import jax
import jax.numpy as jnp
from jax.experimental import pallas as pl


def kernel(x, edge_index, W1, att_src1, att_dst1, bias1, gamma1, beta1, W2, att_src2, att_dst2, bias2):
    raise NotImplementedError("write your pallas kernel here")


# speedup vs baseline: 42.9842x; 42.9842x over previous
import functools

import jax
import jax.numpy as jnp
from jax import lax
from jax.experimental import pallas as pl
from jax.experimental.pallas import tpu as pltpu
from jax.experimental.pallas import tpu_sc as plsc

N = 10000
E = 320000
D = 128
ROW = 144
APAD = 16
NC = 2
NS = 16
NW = NC * NS
EDGES_PER_W = E // NW
K = 80
CHUNKS = EDGES_PER_W // K
NPAD = 10240
ROWS_PER_TILE = NPAD // NS
ZROWS = 128


def _bcast_lane(v, lane):
    idx = jnp.full((16, 1), lane, jnp.int32)
    dn = lax.GatherDimensionNumbers(
        offset_dims=(), collapsed_slice_dims=(0,), start_index_map=(0,))
    return lax.gather(v, idx, dn, (1,),
                      mode=lax.GatherScatterMode.PROMISE_IN_BOUNDS)


def _make_edge_kernel(heads):
    mesh = plsc.VectorSubcoreMesh(core_axis_name="c", subcore_axis_name="s")

    @functools.partial(
        pl.kernel,
        out_type=jax.ShapeDtypeStruct((NC, NPAD, ROW), jnp.float32),
        mesh=mesh,
        compiler_params=pltpu.CompilerParams(use_tc_tiling_on_sc=False),
        scratch_types=[
            pltpu.VMEM((K,), jnp.int32),
            pltpu.VMEM((K,), jnp.int32),
            pltpu.VMEM((K, ROW), jnp.float32),
            pltpu.VMEM((K, APAD), jnp.float32),
            pltpu.VMEM((ZROWS, ROW), jnp.float32),
            pltpu.VMEM_SHARED((NPAD, ROW), jnp.float32),
            pltpu.SemaphoreType.DMA,
            pltpu.SemaphoreType.DMA,
        ],
    )
    def edge_kernel(table, adst_tab, src, dst, out, sidx, didx, rows, adst,
                    zbuf, acc, sem1, sem2):
        cid = lax.axis_index("c")
        sid = lax.axis_index("s")
        wid = cid * NS + sid

        def zero_body(i, carry):
            for j in range(ROW // 16):
                zbuf[i, pl.ds(j * 16, 16)] = jnp.zeros((16,), jnp.float32)
            return carry
        lax.fori_loop(0, ZROWS, zero_body, 0)
        for c in range(ROWS_PER_TILE // ZROWS):
            pltpu.sync_copy(zbuf, acc.at[pl.ds(sid * ROWS_PER_TILE + c * ZROWS,
                                               ZROWS)])
        plsc.subcore_barrier()

        lane = lax.iota(jnp.int32, 16)

        def chunk_body(ci, carry):
            base = wid * EDGES_PER_W + ci * K
            pltpu.sync_copy(src.at[pl.ds(base, K)], sidx)
            pltpu.sync_copy(dst.at[pl.ds(base, K)], didx)
            cp1 = pltpu.async_copy(table.at[sidx], rows, sem1)
            cp2 = pltpu.async_copy(adst_tab.at[didx], adst, sem2)
            cp1.wait()
            cp2.wait()

            def edge_body(k, ecarry):
                asrc = rows[k, pl.ds(D, 16)]
                ad = adst[k, :]
                e = asrc + ad
                e = jnp.where(e > 0.0, e, 0.2 * e)
                w = jnp.exp(e)
                w = jnp.where(lane < heads, w, 0.0)
                rows[k, pl.ds(D, 16)] = w
                if heads == 1:
                    ws = _bcast_lane(w, 0)
                    for g in range(8):
                        rows[k, pl.ds(g * 16, 16)] = (
                            rows[k, pl.ds(g * 16, 16)] * ws)
                else:
                    for g in range(8):
                        ws = _bcast_lane(w, g)
                        rows[k, pl.ds(g * 16, 16)] = (
                            rows[k, pl.ds(g * 16, 16)] * ws)
                return ecarry
            lax.fori_loop(0, K, edge_body, 0)

            pltpu.sync_copy(rows, acc.at[didx], add=True)
            return carry
        lax.fori_loop(0, CHUNKS, chunk_body, 0)

        plsc.subcore_barrier()
        pltpu.sync_copy(acc.at[pl.ds(sid * ROWS_PER_TILE, ROWS_PER_TILE)],
                        out.at[cid, pl.ds(sid * ROWS_PER_TILE, ROWS_PER_TILE)])

    return edge_kernel


_edge_kernel_h8 = _make_edge_kernel(8)
_edge_kernel_h1 = _make_edge_kernel(1)


_BLK = 2000


def _tc_pre_body(x_ref, w_ref, ms_ref, md_ref, t_ref, ad_ref):
    h = jnp.dot(x_ref[...], w_ref[...], preferred_element_type=jnp.float32)
    t_ref[:, pl.ds(0, D)] = h
    t_ref[:, pl.ds(D, APAD)] = jnp.dot(h, ms_ref[...],
                                       preferred_element_type=jnp.float32)
    ad_ref[...] = jnp.dot(h, md_ref[...], preferred_element_type=jnp.float32)


def _tc_mid_body(acc_ref, bias_ref, scale_ref, beta_ref, w2_ref, ms_ref,
                 md_ref, e8_ref, t_ref, ad_ref):
    s = acc_ref[0] + acc_ref[1]
    den = jnp.dot(s[:, D:ROW], e8_ref[...], preferred_element_type=jnp.float32)
    hn = s[:, 0:D] / (den + 1e-16) + bias_ref[...]
    hn = hn * scale_ref[...] + beta_ref[...]
    h = jnp.where(hn > 0.0, hn, jnp.exp(hn) - 1.0)
    h2 = jnp.dot(h, w2_ref[...], preferred_element_type=jnp.float32)
    t_ref[:, pl.ds(0, D)] = h2
    t_ref[:, pl.ds(D, APAD)] = jnp.dot(h2, ms_ref[...],
                                       preferred_element_type=jnp.float32)
    ad_ref[...] = jnp.dot(h2, md_ref[...], preferred_element_type=jnp.float32)


def _tc_post_body(acc_ref, bias_ref, e1_ref, out_ref):
    s = acc_ref[0] + acc_ref[1]
    den = jnp.dot(s[:, D:ROW], e1_ref[...], preferred_element_type=jnp.float32)
    out_ref[...] = s[:, 0:D] / (den + 1e-16) + bias_ref[...]


def _tc_pre(x, w1, ms, md):
    return pl.pallas_call(
        _tc_pre_body,
        grid=(N // _BLK,),
        in_specs=[
            pl.BlockSpec((_BLK, D), lambda i: (i, 0)),
            pl.BlockSpec((D, D), lambda i: (0, 0)),
            pl.BlockSpec((D, APAD), lambda i: (0, 0)),
            pl.BlockSpec((D, APAD), lambda i: (0, 0)),
        ],
        out_specs=[
            pl.BlockSpec((_BLK, ROW), lambda i: (i, 0)),
            pl.BlockSpec((_BLK, APAD), lambda i: (i, 0)),
        ],
        out_shape=[
            jax.ShapeDtypeStruct((N, ROW), jnp.float32),
            jax.ShapeDtypeStruct((N, APAD), jnp.float32),
        ],
    )(x, w1, ms, md)


def _tc_mid(acc, bias, scale, beta, w2, ms, md, e8):
    return pl.pallas_call(
        _tc_mid_body,
        grid=(N // _BLK,),
        in_specs=[
            pl.BlockSpec((NC, _BLK, ROW), lambda i: (0, i, 0)),
            pl.BlockSpec((1, D), lambda i: (0, 0)),
            pl.BlockSpec((1, D), lambda i: (0, 0)),
            pl.BlockSpec((1, D), lambda i: (0, 0)),
            pl.BlockSpec((D, D), lambda i: (0, 0)),
            pl.BlockSpec((D, APAD), lambda i: (0, 0)),
            pl.BlockSpec((D, APAD), lambda i: (0, 0)),
            pl.BlockSpec((APAD, D), lambda i: (0, 0)),
        ],
        out_specs=[
            pl.BlockSpec((_BLK, ROW), lambda i: (i, 0)),
            pl.BlockSpec((_BLK, APAD), lambda i: (i, 0)),
        ],
        out_shape=[
            jax.ShapeDtypeStruct((N, ROW), jnp.float32),
            jax.ShapeDtypeStruct((N, APAD), jnp.float32),
        ],
    )(acc, bias, scale, beta, w2, ms, md, e8)


def _tc_post(acc, bias, e1):
    return pl.pallas_call(
        _tc_post_body,
        grid=(N // _BLK,),
        in_specs=[
            pl.BlockSpec((NC, _BLK, ROW), lambda i: (0, i, 0)),
            pl.BlockSpec((1, D), lambda i: (0, 0)),
            pl.BlockSpec((APAD, D), lambda i: (0, 0)),
        ],
        out_specs=pl.BlockSpec((_BLK, D), lambda i: (i, 0)),
        out_shape=jax.ShapeDtypeStruct((N, D), jnp.float32),
    )(acc, bias, e1)


def _expand_att(att, heads, head_dim):
    eye = jnp.eye(heads, dtype=att.dtype)
    m = att[:, :, None] * eye[:, None, :]
    m = m.reshape(heads * head_dim, heads)
    return jnp.pad(m, ((0, 0), (0, APAD - heads)))


def kernel(x, edge_index, W1, att_src1, att_dst1, bias1, gamma1, beta1,
           W2, att_src2, att_dst2, bias2):
    src = edge_index[0]
    dst = edge_index[1]

    ms1 = _expand_att(att_src1, 8, 16)
    md1 = _expand_att(att_dst1, 8, 16)
    ms2 = _expand_att(att_src2, 1, D)
    md2 = _expand_att(att_dst2, 1, D)
    lane16 = jnp.arange(APAD, dtype=jnp.int32)
    e8 = (lane16[:, None] == (jnp.arange(D, dtype=jnp.int32)[None, :] // 16)
          ).astype(jnp.float32)
    e1 = (lane16[:, None] == 0).astype(jnp.float32) * jnp.ones((1, D),
                                                               jnp.float32)
    scale1 = (gamma1 / jnp.sqrt(1.0 + 1e-5)).reshape(1, D)
    bias1r = bias1.reshape(1, D)
    beta1r = beta1.reshape(1, D)
    bias2r = bias2.reshape(1, D)

    t1, ad1 = _tc_pre(x, W1, ms1, md1)
    acc1 = _edge_kernel_h8(t1, ad1, src, dst)
    t2, ad2 = _tc_mid(acc1, bias1r, scale1, beta1r, W2, ms2, md2, e8)
    acc2 = _edge_kernel_h1(t2, ad2, src, dst)
    return _tc_post(acc2, bias2r, e1)


# speedup vs baseline: 90.9792x; 2.1166x over previous
import functools

import jax
import jax.numpy as jnp
from jax import lax
from jax.experimental import pallas as pl
from jax.experimental.pallas import tpu as pltpu
from jax.experimental.pallas import tpu_sc as plsc

N = 10000
E = 320000
D = 128
ROW = 144
APAD = 16
NC = 2
NS = 16
NW = NC * NS
EDGES_PER_W = E // NW
K = 80
CHUNKS = EDGES_PER_W // K
NPAD = 10240
ROWS_PER_TILE = NPAD // NS
ZROWS = 32


def _bcast_lane(v, lane):
    idx = jnp.full((16, 1), lane, jnp.int32)
    dn = lax.GatherDimensionNumbers(
        offset_dims=(), collapsed_slice_dims=(0,), start_index_map=(0,))
    return lax.gather(v, idx, dn, (1,),
                      mode=lax.GatherScatterMode.PROMISE_IN_BOUNDS)


def _make_edge_kernel(heads):
    mesh = plsc.VectorSubcoreMesh(core_axis_name="c", subcore_axis_name="s")

    @functools.partial(
        pl.kernel,
        out_type=jax.ShapeDtypeStruct((NC, NPAD, ROW), jnp.float32),
        mesh=mesh,
        compiler_params=pltpu.CompilerParams(use_tc_tiling_on_sc=False),
        scratch_types=[
            pltpu.VMEM((2, K), jnp.int32),
            pltpu.VMEM((2, K), jnp.int32),
            pltpu.VMEM((K, ROW), jnp.float32),
            pltpu.VMEM((K, ROW), jnp.float32),
            pltpu.VMEM((K, APAD), jnp.float32),
            pltpu.VMEM((K, APAD), jnp.float32),
            pltpu.VMEM((ZROWS, ROW), jnp.float32),
            pltpu.VMEM_SHARED((NPAD, ROW), jnp.float32),
            pltpu.SemaphoreType.DMA,
            pltpu.SemaphoreType.DMA,
        ],
    )
    def edge_kernel(table, adst_tab, sd4, out, sd_a, sd_b, rows_a,
                    rows_b, adst_a, adst_b, zbuf, acc, sem_a, sem_b):
        cid = lax.axis_index("c")
        sid = lax.axis_index("s")
        wid = cid * NS + sid

        def fetch_idx(ci, sd):
            pltpu.sync_copy(sd4.at[wid, ci], sd)

        def issue(sd, rows, adst, sem):
            pltpu.make_async_copy(table.at[sd.at[0]], rows, sem).start()
            pltpu.make_async_copy(adst_tab.at[sd.at[1]], adst, sem).start()

        def drain(sd, rows, adst, sem):
            pltpu.make_async_copy(table.at[sd.at[0]], rows, sem).wait()
            pltpu.make_async_copy(adst_tab.at[sd.at[1]], adst, sem).wait()

        fetch_idx(0, sd_a)
        issue(sd_a, rows_a, adst_a, sem_a)

        def zero_body(i, carry):
            for j in range(ROW // 16):
                zbuf[i, pl.ds(j * 16, 16)] = jnp.zeros((16,), jnp.float32)
            return carry
        lax.fori_loop(0, ZROWS, zero_body, 0)
        for c in range(ROWS_PER_TILE // ZROWS):
            pltpu.sync_copy(zbuf, acc.at[pl.ds(sid * ROWS_PER_TILE + c * ZROWS,
                                               ZROWS)])
        plsc.subcore_barrier()

        lane = lax.iota(jnp.int32, 16)

        def compute(rows, adst):
            @plsc.parallel_loop(0, K, 1, unroll=4)
            def edge_body(k):
                grp = [rows[k, pl.ds(g * 16, 16)] for g in range(8)]
                asrc = rows[k, pl.ds(D, 16)]
                ad = adst[k, :]
                e = asrc + ad
                e = jnp.where(e > 0.0, e, 0.2 * e)
                w = jnp.exp(e)
                w = jnp.where(lane < heads, w, 0.0)
                rows[k, pl.ds(D, 16)] = w
                if heads == 1:
                    ws = _bcast_lane(w, 0)
                    for g in range(8):
                        rows[k, pl.ds(g * 16, 16)] = grp[g] * ws
                else:
                    for g in range(8):
                        rows[k, pl.ds(g * 16, 16)] = grp[g] * _bcast_lane(w, g)

        def pair_body(j, carry):
            c0 = 2 * j
            fetch_idx(c0 + 1, sd_b)
            issue(sd_b, rows_b, adst_b, sem_b)
            drain(sd_a, rows_a, adst_a, sem_a)
            compute(rows_a, adst_a)
            pltpu.sync_copy(rows_a, acc.at[sd_a.at[1]], add=True)
            fetch_idx(c0 + 2, sd_a)
            issue(sd_a, rows_a, adst_a, sem_a)
            drain(sd_b, rows_b, adst_b, sem_b)
            compute(rows_b, adst_b)
            pltpu.sync_copy(rows_b, acc.at[sd_b.at[1]], add=True)
            return carry
        lax.fori_loop(0, (CHUNKS - 1) // 2, pair_body, 0)
        drain(sd_a, rows_a, adst_a, sem_a)
        compute(rows_a, adst_a)
        pltpu.sync_copy(rows_a, acc.at[sd_a.at[1]], add=True)

        plsc.subcore_barrier()
        pltpu.sync_copy(acc.at[pl.ds(sid * ROWS_PER_TILE, ROWS_PER_TILE)],
                        out.at[cid, pl.ds(sid * ROWS_PER_TILE, ROWS_PER_TILE)])

    return edge_kernel


_edge_kernel_h8 = _make_edge_kernel(8)
_edge_kernel_h1 = _make_edge_kernel(1)


_BLK = 2000


def _tc_pre_body(x_ref, w_ref, ms_ref, md_ref, t_ref, ad_ref):
    h = jnp.dot(x_ref[...], w_ref[...], preferred_element_type=jnp.float32)
    t_ref[:, pl.ds(0, D)] = h
    t_ref[:, pl.ds(D, APAD)] = jnp.dot(h, ms_ref[...],
                                       preferred_element_type=jnp.float32)
    ad_ref[...] = jnp.dot(h, md_ref[...], preferred_element_type=jnp.float32)


def _tc_mid_body(acc_ref, bias_ref, scale_ref, beta_ref, w2_ref, ms_ref,
                 md_ref, e8_ref, t_ref, ad_ref):
    s = acc_ref[0] + acc_ref[1]
    den = jnp.dot(s[:, D:ROW], e8_ref[...], preferred_element_type=jnp.float32)
    hn = s[:, 0:D] / (den + 1e-16) + bias_ref[...]
    hn = hn * scale_ref[...] + beta_ref[...]
    h = jnp.where(hn > 0.0, hn, jnp.exp(hn) - 1.0)
    h2 = jnp.dot(h, w2_ref[...], preferred_element_type=jnp.float32)
    t_ref[:, pl.ds(0, D)] = h2
    t_ref[:, pl.ds(D, APAD)] = jnp.dot(h2, ms_ref[...],
                                       preferred_element_type=jnp.float32)
    ad_ref[...] = jnp.dot(h2, md_ref[...], preferred_element_type=jnp.float32)


def _tc_post_body(acc_ref, bias_ref, e1_ref, out_ref):
    s = acc_ref[0] + acc_ref[1]
    den = jnp.dot(s[:, D:ROW], e1_ref[...], preferred_element_type=jnp.float32)
    out_ref[...] = s[:, 0:D] / (den + 1e-16) + bias_ref[...]


def _tc_pre(x, w1, ms, md):
    return pl.pallas_call(
        _tc_pre_body,
        grid=(N // _BLK,),
        in_specs=[
            pl.BlockSpec((_BLK, D), lambda i: (i, 0)),
            pl.BlockSpec((D, D), lambda i: (0, 0)),
            pl.BlockSpec((D, APAD), lambda i: (0, 0)),
            pl.BlockSpec((D, APAD), lambda i: (0, 0)),
        ],
        out_specs=[
            pl.BlockSpec((_BLK, ROW), lambda i: (i, 0)),
            pl.BlockSpec((_BLK, APAD), lambda i: (i, 0)),
        ],
        out_shape=[
            jax.ShapeDtypeStruct((N, ROW), jnp.float32),
            jax.ShapeDtypeStruct((N, APAD), jnp.float32),
        ],
    )(x, w1, ms, md)


def _tc_mid(acc, bias, scale, beta, w2, ms, md, e8):
    return pl.pallas_call(
        _tc_mid_body,
        grid=(N // _BLK,),
        in_specs=[
            pl.BlockSpec((NC, _BLK, ROW), lambda i: (0, i, 0)),
            pl.BlockSpec((1, D), lambda i: (0, 0)),
            pl.BlockSpec((1, D), lambda i: (0, 0)),
            pl.BlockSpec((1, D), lambda i: (0, 0)),
            pl.BlockSpec((D, D), lambda i: (0, 0)),
            pl.BlockSpec((D, APAD), lambda i: (0, 0)),
            pl.BlockSpec((D, APAD), lambda i: (0, 0)),
            pl.BlockSpec((APAD, D), lambda i: (0, 0)),
        ],
        out_specs=[
            pl.BlockSpec((_BLK, ROW), lambda i: (i, 0)),
            pl.BlockSpec((_BLK, APAD), lambda i: (i, 0)),
        ],
        out_shape=[
            jax.ShapeDtypeStruct((N, ROW), jnp.float32),
            jax.ShapeDtypeStruct((N, APAD), jnp.float32),
        ],
    )(acc, bias, scale, beta, w2, ms, md, e8)


def _tc_post(acc, bias, e1):
    return pl.pallas_call(
        _tc_post_body,
        grid=(N // _BLK,),
        in_specs=[
            pl.BlockSpec((NC, _BLK, ROW), lambda i: (0, i, 0)),
            pl.BlockSpec((1, D), lambda i: (0, 0)),
            pl.BlockSpec((APAD, D), lambda i: (0, 0)),
        ],
        out_specs=pl.BlockSpec((_BLK, D), lambda i: (i, 0)),
        out_shape=jax.ShapeDtypeStruct((N, D), jnp.float32),
    )(acc, bias, e1)


def _expand_att(att, heads, head_dim):
    eye = jnp.eye(heads, dtype=att.dtype)
    m = att[:, :, None] * eye[:, None, :]
    m = m.reshape(heads * head_dim, heads)
    return jnp.pad(m, ((0, 0), (0, APAD - heads)))


def kernel(x, edge_index, W1, att_src1, att_dst1, bias1, gamma1, beta1,
           W2, att_src2, att_dst2, bias2):
    sd4 = jnp.stack([edge_index[0].reshape(NW, CHUNKS, K),
                     edge_index[1].reshape(NW, CHUNKS, K)], axis=2)

    ms1 = _expand_att(att_src1, 8, 16)
    md1 = _expand_att(att_dst1, 8, 16)
    ms2 = _expand_att(att_src2, 1, D)
    md2 = _expand_att(att_dst2, 1, D)
    lane16 = jnp.arange(APAD, dtype=jnp.int32)
    e8 = (lane16[:, None] == (jnp.arange(D, dtype=jnp.int32)[None, :] // 16)
          ).astype(jnp.float32)
    e1 = (lane16[:, None] == 0).astype(jnp.float32) * jnp.ones((1, D),
                                                               jnp.float32)
    scale1 = (gamma1 / jnp.sqrt(1.0 + 1e-5)).reshape(1, D)
    bias1r = bias1.reshape(1, D)
    beta1r = beta1.reshape(1, D)
    bias2r = bias2.reshape(1, D)

    t1, ad1 = _tc_pre(x, W1, ms1, md1)
    acc1 = _edge_kernel_h8(t1, ad1, sd4)
    t2, ad2 = _tc_mid(acc1, bias1r, scale1, beta1r, W2, ms2, md2, e8)
    acc2 = _edge_kernel_h1(t2, ad2, sd4)
    return _tc_post(acc2, bias2r, e1)
